# baseline (speedup 1.0000x reference)
import functools

import jax
import jax.numpy as jnp
from jax import lax
from jax.experimental import pallas as pl
from jax.experimental.pallas import tpu as pltpu
from jax.experimental.pallas import tpu_sc as plsc

B, L, D = 4096, 50, 32
N_ITEMS_PAD, N_TIME_PAD, N_RAT_PAD = 100096, 1024, 64
G = 8
DP = G * D
NW = 32
UPW = B // NW
BG = B // G
SBU = 64
NB = BG // SBU
EPS = 1e-5


def _pretransform_body(i2e, t2e, r2e, bd_i, bd_t, bd_r, bias_p,
                       o_i, o_t, o_r):
    f32 = jnp.float32
    bf16 = jnp.bfloat16
    vi = jnp.dot(i2e[...], bd_i[...], preferred_element_type=f32).astype(bf16)
    o_i[...] = jnp.concatenate([vi, jnp.zeros((24, 128), bf16)], axis=0)
    o_t[...] = jnp.dot(t2e[...], bd_t[...], preferred_element_type=f32).astype(bf16)
    o_r[...] = (jnp.dot(r2e[...], bd_r[...], preferred_element_type=f32)
                + bias_p[...]).astype(bf16)


def _pretransform(i2e_p, t2e_p, r2e_p, bd_i, bd_t, bd_r, bias_p):
    return pl.pallas_call(
        _pretransform_body,
        out_shape=[
            jax.ShapeDtypeStruct((N_ITEMS_PAD * D // 128, 128), jnp.bfloat16),
            jax.ShapeDtypeStruct(t2e_p.shape, jnp.bfloat16),
            jax.ShapeDtypeStruct(r2e_p.shape, jnp.bfloat16),
        ],
    )(i2e_p, t2e_p, r2e_p, bd_i, bd_t, bd_r, bias_p)


def _sc_gather_body(item_slab, rat_slab, time_slab, uidx_slab,
                    i2e_t, t2e_t, r2e_t, u2e,
                    x1_out, u_out,
                    ix0_i, ix0_t, ix0_r, ix1_i, ix1_t, ix1_r, uidx_v,
                    s_item, s_time, s_rat,
                    ri0, rt0, rr0, ra0, ri1, rt1, rr1, ra1, u_rows,
                    sem_g0, sem_g1, sem_w0, sem_w1, sem_i0, sem_i1):
    wid = lax.axis_index("s") * 2 + lax.axis_index("c")
    b0 = wid * UPW

    @pl.when(lax.axis_index("s") == 0)
    def _stage():
        pltpu.sync_copy(i2e_t, s_item)
        pltpu.sync_copy(t2e_t, s_time)
        pltpu.sync_copy(r2e_t, s_rat)

    pltpu.sync_copy(uidx_slab.at[wid], uidx_v)
    pltpu.async_copy(u2e.at[uidx_v], u_rows, sem_g0).wait()
    pltpu.sync_copy(u_rows, u_out.at[pl.ds(b0, UPW)])

    plsc.subcore_barrier()

    def idx_fetch(l, ix_i, ix_t, ix_r, sem_i):
        pltpu.async_copy(item_slab.at[wid * L + l], ix_i, sem_i)
        pltpu.async_copy(time_slab.at[wid * L + l], ix_t, sem_i)
        pltpu.async_copy(rat_slab.at[wid * L + l], ix_r, sem_i)

    def drain_idx(ix_i, sem_i):
        for _ in range(3):
            pltpu.make_async_copy(item_slab.at[0], ix_i, sem_i).wait()

    def issue(l, ix_i, ix_t, ix_r, ri, rt, rr, sem_g, sem_i):
        drain_idx(ix_i, sem_i)
        pltpu.async_copy(s_item.at[ix_i], ri, sem_g)
        pltpu.async_copy(s_time.at[ix_t], rt, sem_g)
        pltpu.async_copy(s_rat.at[ix_r], rr, sem_g)

    def drain(dst_ref, sem, n=1):
        for _ in range(n):
            pltpu.make_async_copy(x1_out.at[0, pl.ds(b0, UPW)], dst_ref, sem).wait()

    def drain_w(src_ref, sem):
        pltpu.make_async_copy(src_ref, x1_out.at[0, pl.ds(b0, UPW)], sem).wait()

    def process(l, ix_i, ix_t, ix_r, ri, rt, rr, ra, sem_g, sem_w, sem_i):
        drain(ri, sem_g, 3)

        @pl.when(l + 2 < L)
        def _():
            idx_fetch(l + 2, ix_i, ix_t, ix_r, sem_i)

        def _add(r, c2):
            ra[r, :] = ri[r, :] + rt[r, :] + rr[r, :]
            return c2

        lax.fori_loop(0, UPW, _add, 0, unroll=8)

        pltpu.async_copy(ra, x1_out.at[l, pl.ds(b0, UPW)], sem_w)

    buf0 = (ix0_i, ix0_t, ix0_r, ri0, rt0, rr0)
    buf1 = (ix1_i, ix1_t, ix1_r, ri1, rt1, rr1)
    idx_fetch(0, ix0_i, ix0_t, ix0_r, sem_i0)
    idx_fetch(1, ix1_i, ix1_t, ix1_r, sem_i1)
    issue(0, *buf0, sem_g0, sem_i0)

    def outer(g, c):
        l0 = 2 * g

        @pl.when(g > 0)
        def _():
            drain_w(ra1, sem_w1)

        issue(l0 + 1, *buf1, sem_g1, sem_i1)
        process(l0, *buf0, ra0, sem_g0, sem_w0, sem_i0)
        drain_w(ra0, sem_w0)

        @pl.when(l0 + 2 < L)
        def _():
            issue(l0 + 2, *buf0, sem_g0, sem_i0)

        process(l0 + 1, *buf1, ra1, sem_g1, sem_w1, sem_i1)
        return c

    lax.fori_loop(0, L // 2, outer, 0)
    drain_w(ra1, sem_w1)


def _sc_gather(item_slab, rat_slab, time_slab, uidx_slab, i2e_t, t2e_t, r2e_t, u2e):
    mesh = plsc.VectorSubcoreMesh(core_axis_name="c", subcore_axis_name="s")
    f = pl.kernel(
        _sc_gather_body,
        out_type=[
            jax.ShapeDtypeStruct((L, B, D), jnp.bfloat16),
            jax.ShapeDtypeStruct((B, D), jnp.float32),
        ],
        mesh=mesh,
        scratch_types=[
            pltpu.VMEM((UPW,), jnp.int32),
            pltpu.VMEM((UPW,), jnp.int32),
            pltpu.VMEM((UPW,), jnp.int32),
            pltpu.VMEM((UPW,), jnp.int32),
            pltpu.VMEM((UPW,), jnp.int32),
            pltpu.VMEM((UPW,), jnp.int32),
            pltpu.VMEM((UPW,), jnp.int32),
            pltpu.VMEM_SHARED((N_ITEMS_PAD, D), jnp.bfloat16),
            pltpu.VMEM_SHARED((N_TIME_PAD, D), jnp.bfloat16),
            pltpu.VMEM_SHARED((N_RAT_PAD, D), jnp.bfloat16),
            pltpu.VMEM((UPW, D), jnp.bfloat16),
            pltpu.VMEM((UPW, D), jnp.bfloat16),
            pltpu.VMEM((UPW, D), jnp.bfloat16),
            pltpu.VMEM((UPW, D), jnp.bfloat16),
            pltpu.VMEM((UPW, D), jnp.bfloat16),
            pltpu.VMEM((UPW, D), jnp.bfloat16),
            pltpu.VMEM((UPW, D), jnp.bfloat16),
            pltpu.VMEM((UPW, D), jnp.bfloat16),
            pltpu.VMEM((UPW, D), jnp.float32),
            pltpu.SemaphoreType.DMA,
            pltpu.SemaphoreType.DMA,
            pltpu.SemaphoreType.DMA,
            pltpu.SemaphoreType.DMA,
            pltpu.SemaphoreType.DMA,
            pltpu.SemaphoreType.DMA,
        ],
        compiler_params=pltpu.CompilerParams(use_tc_tiling_on_sc=False),
    )
    return f(item_slab, rat_slab, time_slab, uidx_slab, i2e_t, t2e_t, r2e_t, u2e)


def _main_body(x1, up, bd_e1, be1, bd_a1x, bd_a1u, ba1, bd_a2, ba2,
               e3, r8, bd_lu, bd_la, blin, e1, w1b, g1, b1,
               e2, w2b, k1, k1t, g2, b2, e4, k2, k2t, w3b,
               out, h1s):
    i = pl.program_id(0)
    f32 = jnp.float32

    def mm(a, b):
        return jnp.dot(a, b, preferred_element_type=f32)

    xf = jnp.maximum(x1[...].reshape(L * SBU, DP).astype(f32), 0.0)
    u = up[...].reshape(SBU, DP)
    x2 = jnp.maximum(mm(xf, bd_e1[...]) + be1[...], 0.0)
    uc = mm(u, bd_a1u[...]) + ba1[...]
    a1 = mm(x2, bd_a1x[...]).reshape(L, SBU, DP) + uc[None]
    a1 = jnp.maximum(a1, 0.0).reshape(L * SBU, DP)
    a2 = jnp.maximum(mm(a1, bd_a2[...]) + ba2[...], 0.0)
    s = mm(a2, e3[...]).reshape(L, SBU, G)
    smax = jnp.max(s, axis=0, keepdims=True)
    e = jnp.exp(s - smax)
    att = e / jnp.sum(e, axis=0, keepdims=True)
    attx = mm(att.reshape(L * SBU, G), r8[...]).reshape(L, SBU, DP)
    all_emb = jnp.sum(attx * x2.reshape(L, SBU, DP), axis=0)
    comb = jnp.maximum(mm(all_emb, bd_la[...]) + mm(u, bd_lu[...]) + blin[...], 0.0)
    h1s[pl.ds(i * SBU, SBU), :] = mm(comb, e1[...]) + w1b[...]

    @pl.when(i == NB - 1)
    def _tail():
        hp = h1s[...]
        s1 = jnp.sum(hp, axis=0, keepdims=True)
        m8 = mm(s1, k1[...]) * (1.0 / B)
        mp = mm(m8, k1t[...])
        d = hp - mp
        v8 = mm(jnp.sum(d * d, axis=0, keepdims=True), k1[...]) * (1.0 / B)
        vp = mm(v8, k1t[...])
        h1 = jnp.maximum(d * lax.rsqrt(vp + EPS) * g1[...] + b1[...], 0.0)
        z2 = mm(h1, e2[...]) + w2b[...]
        s2 = jnp.sum(z2, axis=0, keepdims=True)
        m4 = mm(s2, k2[...]) * (1.0 / B)
        mp2 = mm(m4, k2t[...])
        d2 = z2 - mp2
        v4 = mm(jnp.sum(d2 * d2, axis=0, keepdims=True), k2[...]) * (1.0 / B)
        vp2 = mm(v4, k2t[...])
        h2 = jnp.maximum(d2 * lax.rsqrt(vp2 + EPS) * g2[...] + b2[...], 0.0)
        out[...] = mm(h2, e4[...]) + w3b[...]


def _main(x1_p, u_p, weights):
    full = lambda shape: pl.BlockSpec(shape, lambda i: tuple(0 for _ in shape))
    in_specs = [
        pl.BlockSpec((L, SBU * 2, 128), lambda i: (0, i, 0)),
        pl.BlockSpec((SBU * 2, 128), lambda i: (i, 0)),
    ] + [full(w.shape) for w in weights]
    return pl.pallas_call(
        _main_body,
        grid=(NB,),
        in_specs=in_specs,
        out_specs=pl.BlockSpec((BG, G), lambda i: (0, 0)),
        out_shape=jax.ShapeDtypeStruct((BG, G), jnp.float32),
        scratch_shapes=[pltpu.VMEM((BG, G * 8), jnp.float32)],
    )(x1_p, u_p, *weights)


def kernel(user_idx, item_neigh, rat_neigh, time_neigh, u2e_w, i2e_w, r2e_w, t2e_w,
           neigh_e_W, neigh_e_b, neigh_e1_W, neigh_e1_b,
           att_W1, att_b1, att_W2, att_b2, att_W3, att_b3,
           linear_W, linear_b, w1_W, w1_b, bn1_g, bn1_b,
           w2_W, w2_b, bn2_g, bn2_b, w3_W, w3_b):
    f32 = jnp.float32
    eye8 = jnp.eye(G, dtype=f32)
    kron = jnp.kron

    eye4 = jnp.eye(4, dtype=f32)
    i2e_p = i2e_w.astype(f32).reshape(-1, 128)
    t2e_p = jnp.pad(t2e_w.astype(f32), ((0, N_TIME_PAD - t2e_w.shape[0]), (0, 0))).reshape(-1, 128)
    r2e_p = jnp.pad(r2e_w.astype(f32), ((0, N_RAT_PAD - r2e_w.shape[0]), (0, 0))).reshape(-1, 128)
    bd_i = kron(eye4, neigh_e_W[:, 0 * D:1 * D].T.astype(f32))
    bd_r = kron(eye4, neigh_e_W[:, 1 * D:2 * D].T.astype(f32))
    bd_t = kron(eye4, neigh_e_W[:, 2 * D:3 * D].T.astype(f32))
    bias_p = jnp.tile(neigh_e_b.astype(f32), 4).reshape(1, 128)
    i2e_t, t2e_t, r2e_t = _pretransform(i2e_p, t2e_p, r2e_p, bd_i, bd_t, bd_r, bias_p)
    i2e_t = i2e_t.reshape(-1, D)
    t2e_t = t2e_t.reshape(-1, D)
    r2e_t = r2e_t.reshape(-1, D)

    def slab(idx):
        return (idx.astype(jnp.int32).T.reshape(L, NW, UPW)
                .transpose(1, 0, 2).reshape(NW * L, UPW))

    x1, u = _sc_gather(slab(item_neigh), slab(rat_neigh), slab(time_neigh),
                       user_idx.astype(jnp.int32).reshape(NW, UPW),
                       i2e_t, t2e_t, r2e_t, u2e_w.astype(f32))

    x1_p = x1.reshape(L, B * D // 128, 128)
    u_p = u.reshape(B * D // 128, 128)
    tile = lambda v, n: jnp.tile(v.astype(f32), n).reshape(1, -1)
    weights = [
        kron(eye8, neigh_e1_W.T.astype(f32)), tile(neigh_e1_b, G),
        kron(eye8, att_W1[:, D:].T.astype(f32)),
        kron(eye8, att_W1[:, :D].T.astype(f32)), tile(att_b1, G),
        kron(eye8, att_W2.T.astype(f32)), tile(att_b2, G),
        kron(eye8, att_W3.T.astype(f32)),
        kron(eye8, jnp.ones((1, D), f32)),
        kron(eye8, linear_W[:, :D].T.astype(f32)),
        kron(eye8, linear_W[:, D:].T.astype(f32)), tile(linear_b, G),
        kron(eye8, w1_W.T.astype(f32)), tile(w1_b, G),
        tile(bn1_g, G), tile(bn1_b, G),
        kron(eye8, w2_W.T.astype(f32)), tile(w2_b, G),
        kron(jnp.ones((G, 1), f32), jnp.eye(8, dtype=f32)),
        kron(jnp.ones((1, G), f32), jnp.eye(8, dtype=f32)),
        tile(bn2_g, G), tile(bn2_b, G),
        kron(eye8, w3_W.T.astype(f32)),
        kron(jnp.ones((G, 1), f32), jnp.eye(4, dtype=f32)),
        kron(jnp.ones((1, G), f32), jnp.eye(4, dtype=f32)),
        w3_b.astype(f32).reshape(1, 1),
    ]
    out = _main(x1_p, u_p, weights)
    return out.reshape(B)


# speedup vs baseline: 1.3380x; 1.3380x over previous
import functools

import jax
import jax.numpy as jnp
from jax import lax
from jax.experimental import pallas as pl
from jax.experimental.pallas import tpu as pltpu
from jax.experimental.pallas import tpu_sc as plsc

B, L, D = 4096, 50, 32
N_ITEMS_PAD, N_TIME_PAD, N_RAT_PAD = 100096, 1024, 64
G = 8
DP = G * D
NW = 32
UPW = B // NW
BG = B // G
SBU = 64
NB = BG // SBU
EPS = 1e-5


def _pretransform_body(i2e, t2e, r2e, bd_i, bd_t, bd_r, bias_p,
                       o_i, o_t, o_r):
    f32 = jnp.float32
    bf16 = jnp.bfloat16
    vi = jnp.dot(i2e[...], bd_i[...], preferred_element_type=f32).astype(bf16)
    o_i[...] = jnp.concatenate([vi, jnp.zeros((24, 128), bf16)], axis=0)
    o_t[...] = jnp.dot(t2e[...], bd_t[...], preferred_element_type=f32).astype(bf16)
    o_r[...] = (jnp.dot(r2e[...], bd_r[...], preferred_element_type=f32)
                + bias_p[...]).astype(bf16)


def _pretransform(i2e_p, t2e_p, r2e_p, bd_i, bd_t, bd_r, bias_p):
    return pl.pallas_call(
        _pretransform_body,
        out_shape=[
            jax.ShapeDtypeStruct((N_ITEMS_PAD * D // 128, 128), jnp.bfloat16),
            jax.ShapeDtypeStruct(t2e_p.shape, jnp.bfloat16),
            jax.ShapeDtypeStruct(r2e_p.shape, jnp.bfloat16),
        ],
    )(i2e_p, t2e_p, r2e_p, bd_i, bd_t, bd_r, bias_p)


def _sc_gather_body(item_slab, rat_slab, time_slab, uidx_slab,
                    i2e_t, t2e_t, r2e_t, u2e,
                    x1_out, u_out,
                    ix0_i, ix0_t, ix0_r, ix1_i, ix1_t, ix1_r, uidx_v,
                    s_item, s_time, s_rat,
                    ri0, rt0, rr0, ra0, ri1, rt1, rr1, ra1, u_rows,
                    sem_g0, sem_g1, sem_w0, sem_w1, sem_i0, sem_i1):
    wid = lax.axis_index("s") * 2 + lax.axis_index("c")
    b0 = wid * UPW

    @pl.when(lax.axis_index("s") == 0)
    def _stage():
        pltpu.sync_copy(i2e_t, s_item)
        pltpu.sync_copy(t2e_t, s_time)
        pltpu.sync_copy(r2e_t, s_rat)

    pltpu.sync_copy(uidx_slab.at[wid], uidx_v)
    pltpu.async_copy(u2e.at[uidx_v], u_rows, sem_g0).wait()
    pltpu.sync_copy(u_rows, u_out.at[pl.ds(b0, UPW)])

    plsc.subcore_barrier()

    def idx_fetch(l, ix_i, ix_t, ix_r, sem_i):
        pltpu.async_copy(item_slab.at[wid * L + l], ix_i, sem_i)
        pltpu.async_copy(time_slab.at[wid * L + l], ix_t, sem_i)
        pltpu.async_copy(rat_slab.at[wid * L + l], ix_r, sem_i)

    def drain_idx(ix_i, sem_i):
        for _ in range(3):
            pltpu.make_async_copy(item_slab.at[0], ix_i, sem_i).wait()

    def issue(l, ix_i, ix_t, ix_r, ri, rt, rr, sem_g, sem_i):
        drain_idx(ix_i, sem_i)
        pltpu.async_copy(s_item.at[ix_i], ri, sem_g)
        pltpu.async_copy(s_time.at[ix_t], rt, sem_g)
        pltpu.async_copy(s_rat.at[ix_r], rr, sem_g)

    def drain(dst_ref, sem, n=1):
        for _ in range(n):
            pltpu.make_async_copy(x1_out.at[0, pl.ds(wid * 32, 32)], dst_ref, sem).wait()

    def drain_w(src_ref, sem):
        pltpu.make_async_copy(src_ref, x1_out.at[0, pl.ds(wid * 32, 32)], sem).wait()

    def process(l, ix_i, ix_t, ix_r, ri, rt, rr, ra, sem_g, sem_w, sem_i):
        drain(ri, sem_g, 3)

        @pl.when(l + 2 < L)
        def _():
            idx_fetch(l + 2, ix_i, ix_t, ix_r, sem_i)

        def _add(r, c2):
            ra[r // 4, pl.ds((r % 4) * D, D)] = ri[r, :] + rt[r, :] + rr[r, :]
            return c2

        lax.fori_loop(0, UPW, _add, 0, unroll=8)

        pltpu.async_copy(ra, x1_out.at[l, pl.ds(wid * 32, 32)], sem_w)

    buf0 = (ix0_i, ix0_t, ix0_r, ri0, rt0, rr0)
    buf1 = (ix1_i, ix1_t, ix1_r, ri1, rt1, rr1)
    idx_fetch(0, ix0_i, ix0_t, ix0_r, sem_i0)
    idx_fetch(1, ix1_i, ix1_t, ix1_r, sem_i1)
    issue(0, *buf0, sem_g0, sem_i0)

    def outer(g, c):
        l0 = 2 * g

        @pl.when(g > 0)
        def _():
            drain_w(ra1, sem_w1)

        issue(l0 + 1, *buf1, sem_g1, sem_i1)
        process(l0, *buf0, ra0, sem_g0, sem_w0, sem_i0)
        drain_w(ra0, sem_w0)

        @pl.when(l0 + 2 < L)
        def _():
            issue(l0 + 2, *buf0, sem_g0, sem_i0)

        process(l0 + 1, *buf1, ra1, sem_g1, sem_w1, sem_i1)
        return c

    lax.fori_loop(0, L // 2, outer, 0)
    drain_w(ra1, sem_w1)


def _sc_gather(item_slab, rat_slab, time_slab, uidx_slab, i2e_t, t2e_t, r2e_t, u2e):
    mesh = plsc.VectorSubcoreMesh(core_axis_name="c", subcore_axis_name="s")
    f = pl.kernel(
        _sc_gather_body,
        out_type=[
            jax.ShapeDtypeStruct((L, B * D // 128, 128), jnp.bfloat16),
            jax.ShapeDtypeStruct((B, D), jnp.float32),
        ],
        mesh=mesh,
        scratch_types=[
            pltpu.VMEM((UPW,), jnp.int32),
            pltpu.VMEM((UPW,), jnp.int32),
            pltpu.VMEM((UPW,), jnp.int32),
            pltpu.VMEM((UPW,), jnp.int32),
            pltpu.VMEM((UPW,), jnp.int32),
            pltpu.VMEM((UPW,), jnp.int32),
            pltpu.VMEM((UPW,), jnp.int32),
            pltpu.VMEM_SHARED((N_ITEMS_PAD, D), jnp.bfloat16),
            pltpu.VMEM_SHARED((N_TIME_PAD, D), jnp.bfloat16),
            pltpu.VMEM_SHARED((N_RAT_PAD, D), jnp.bfloat16),
            pltpu.VMEM((UPW, D), jnp.bfloat16),
            pltpu.VMEM((UPW, D), jnp.bfloat16),
            pltpu.VMEM((UPW, D), jnp.bfloat16),
            pltpu.VMEM((UPW * D // 128, 128), jnp.bfloat16),
            pltpu.VMEM((UPW, D), jnp.bfloat16),
            pltpu.VMEM((UPW, D), jnp.bfloat16),
            pltpu.VMEM((UPW, D), jnp.bfloat16),
            pltpu.VMEM((UPW * D // 128, 128), jnp.bfloat16),
            pltpu.VMEM((UPW, D), jnp.float32),
            pltpu.SemaphoreType.DMA,
            pltpu.SemaphoreType.DMA,
            pltpu.SemaphoreType.DMA,
            pltpu.SemaphoreType.DMA,
            pltpu.SemaphoreType.DMA,
            pltpu.SemaphoreType.DMA,
        ],
        compiler_params=pltpu.CompilerParams(use_tc_tiling_on_sc=False),
    )
    return f(item_slab, rat_slab, time_slab, uidx_slab, i2e_t, t2e_t, r2e_t, u2e)


def _main_body(x1, up, bd_e1, be1, bd_a1x, bd_a1u, ba1, bd_a2, ba2,
               e3, r8, bd_lu, bd_la, blin, e1, w1b, g1, b1,
               e2, w2b, k1, k1t, g2, b2, e4, k2, k2t, w3b,
               out, h1s):
    i = pl.program_id(0)
    f32 = jnp.float32

    def mm(a, b):
        return jnp.dot(a, b, preferred_element_type=f32)

    xf = jnp.maximum(x1[...].reshape(L * SBU, DP).astype(f32), 0.0)
    u = up[...].reshape(SBU, DP)
    x2 = jnp.maximum(mm(xf, bd_e1[...]) + be1[...], 0.0)
    uc = mm(u, bd_a1u[...]) + ba1[...]
    a1 = mm(x2, bd_a1x[...]).reshape(L, SBU, DP) + uc[None]
    a1 = jnp.maximum(a1, 0.0).reshape(L * SBU, DP)
    a2 = jnp.maximum(mm(a1, bd_a2[...]) + ba2[...], 0.0)
    s = mm(a2, e3[...]).reshape(L, SBU, G)
    smax = jnp.max(s, axis=0, keepdims=True)
    e = jnp.exp(s - smax)
    att = e / jnp.sum(e, axis=0, keepdims=True)
    attx = mm(att.reshape(L * SBU, G), r8[...]).reshape(L, SBU, DP)
    all_emb = jnp.sum(attx * x2.reshape(L, SBU, DP), axis=0)
    comb = jnp.maximum(mm(all_emb, bd_la[...]) + mm(u, bd_lu[...]) + blin[...], 0.0)
    h1s[pl.ds(i * SBU, SBU), :] = mm(comb, e1[...]) + w1b[...]

    @pl.when(i == NB - 1)
    def _tail():
        hp = h1s[...]
        s1 = jnp.sum(hp, axis=0, keepdims=True)
        m8 = mm(s1, k1[...]) * (1.0 / B)
        mp = mm(m8, k1t[...])
        d = hp - mp
        v8 = mm(jnp.sum(d * d, axis=0, keepdims=True), k1[...]) * (1.0 / B)
        vp = mm(v8, k1t[...])
        h1 = jnp.maximum(d * lax.rsqrt(vp + EPS) * g1[...] + b1[...], 0.0)
        z2 = mm(h1, e2[...]) + w2b[...]
        s2 = jnp.sum(z2, axis=0, keepdims=True)
        m4 = mm(s2, k2[...]) * (1.0 / B)
        mp2 = mm(m4, k2t[...])
        d2 = z2 - mp2
        v4 = mm(jnp.sum(d2 * d2, axis=0, keepdims=True), k2[...]) * (1.0 / B)
        vp2 = mm(v4, k2t[...])
        h2 = jnp.maximum(d2 * lax.rsqrt(vp2 + EPS) * g2[...] + b2[...], 0.0)
        out[...] = mm(h2, e4[...]) + w3b[...]


def _main(x1_p, u_p, weights):
    full = lambda shape: pl.BlockSpec(shape, lambda i: tuple(0 for _ in shape))
    in_specs = [
        pl.BlockSpec((L, SBU * 2, 128), lambda i: (0, i, 0)),
        pl.BlockSpec((SBU * 2, 128), lambda i: (i, 0)),
    ] + [full(w.shape) for w in weights]
    return pl.pallas_call(
        _main_body,
        grid=(NB,),
        in_specs=in_specs,
        out_specs=pl.BlockSpec((BG, G), lambda i: (0, 0)),
        out_shape=jax.ShapeDtypeStruct((BG, G), jnp.float32),
        scratch_shapes=[pltpu.VMEM((BG, G * 8), jnp.float32)],
    )(x1_p, u_p, *weights)


def kernel(user_idx, item_neigh, rat_neigh, time_neigh, u2e_w, i2e_w, r2e_w, t2e_w,
           neigh_e_W, neigh_e_b, neigh_e1_W, neigh_e1_b,
           att_W1, att_b1, att_W2, att_b2, att_W3, att_b3,
           linear_W, linear_b, w1_W, w1_b, bn1_g, bn1_b,
           w2_W, w2_b, bn2_g, bn2_b, w3_W, w3_b):
    f32 = jnp.float32
    eye8 = jnp.eye(G, dtype=f32)
    kron = jnp.kron

    eye4 = jnp.eye(4, dtype=f32)
    i2e_p = i2e_w.astype(f32).reshape(-1, 128)
    t2e_p = jnp.pad(t2e_w.astype(f32), ((0, N_TIME_PAD - t2e_w.shape[0]), (0, 0))).reshape(-1, 128)
    r2e_p = jnp.pad(r2e_w.astype(f32), ((0, N_RAT_PAD - r2e_w.shape[0]), (0, 0))).reshape(-1, 128)
    bd_i = kron(eye4, neigh_e_W[:, 0 * D:1 * D].T.astype(f32))
    bd_r = kron(eye4, neigh_e_W[:, 1 * D:2 * D].T.astype(f32))
    bd_t = kron(eye4, neigh_e_W[:, 2 * D:3 * D].T.astype(f32))
    bias_p = jnp.tile(neigh_e_b.astype(f32), 4).reshape(1, 128)
    i2e_t, t2e_t, r2e_t = _pretransform(i2e_p, t2e_p, r2e_p, bd_i, bd_t, bd_r, bias_p)
    i2e_t = i2e_t.reshape(-1, D)
    t2e_t = t2e_t.reshape(-1, D)
    r2e_t = r2e_t.reshape(-1, D)

    def slab(idx):
        return (idx.astype(jnp.int32).T.reshape(L, NW, UPW)
                .transpose(1, 0, 2).reshape(NW * L, UPW))

    x1, u = _sc_gather(slab(item_neigh), slab(rat_neigh), slab(time_neigh),
                       user_idx.astype(jnp.int32).reshape(NW, UPW),
                       i2e_t, t2e_t, r2e_t, u2e_w.astype(f32))

    x1_p = x1
    u_p = u.reshape(B * D // 128, 128)
    tile = lambda v, n: jnp.tile(v.astype(f32), n).reshape(1, -1)
    weights = [
        kron(eye8, neigh_e1_W.T.astype(f32)), tile(neigh_e1_b, G),
        kron(eye8, att_W1[:, D:].T.astype(f32)),
        kron(eye8, att_W1[:, :D].T.astype(f32)), tile(att_b1, G),
        kron(eye8, att_W2.T.astype(f32)), tile(att_b2, G),
        kron(eye8, att_W3.T.astype(f32)),
        kron(eye8, jnp.ones((1, D), f32)),
        kron(eye8, linear_W[:, :D].T.astype(f32)),
        kron(eye8, linear_W[:, D:].T.astype(f32)), tile(linear_b, G),
        kron(eye8, w1_W.T.astype(f32)), tile(w1_b, G),
        tile(bn1_g, G), tile(bn1_b, G),
        kron(eye8, w2_W.T.astype(f32)), tile(w2_b, G),
        kron(jnp.ones((G, 1), f32), jnp.eye(8, dtype=f32)),
        kron(jnp.ones((1, G), f32), jnp.eye(8, dtype=f32)),
        tile(bn2_g, G), tile(bn2_b, G),
        kron(eye8, w3_W.T.astype(f32)),
        kron(jnp.ones((G, 1), f32), jnp.eye(4, dtype=f32)),
        kron(jnp.ones((1, G), f32), jnp.eye(4, dtype=f32)),
        w3_b.astype(f32).reshape(1, 1),
    ]
    out = _main(x1_p, u_p, weights)
    return out.reshape(B)
